# baseline (speedup 1.0000x reference)
import jax
import jax.numpy as jnp
from jax.experimental import pallas as pl
from jax.experimental.pallas import tpu as pltpu

_BR = 512


def _s1_body(x_ref, w1_ref, o_ref):
    o_ref[...] = jnp.dot(
        x_ref[...].astype(jnp.bfloat16),
        w1_ref[...].astype(jnp.bfloat16),
        preferred_element_type=jnp.float32,
    ).astype(jnp.bfloat16)


def _layer1_body(adj_ref, s1_ref, b1_ref, w2_ref, g_ref):
    a = adj_ref[...].astype(jnp.bfloat16)
    h = jnp.dot(a, s1_ref[...], preferred_element_type=jnp.float32)
    h = jnp.maximum(h + b1_ref[...], 0.0)
    g_ref[...] = jnp.dot(
        h.astype(jnp.bfloat16), w2_ref[...], preferred_element_type=jnp.float32
    ).astype(jnp.bfloat16)


def _layer2_body(adj_ref, g_ref, b2_ref, o_ref):
    a = adj_ref[...].astype(jnp.bfloat16)
    o_ref[...] = (
        jnp.dot(a, g_ref[...], preferred_element_type=jnp.float32) + b2_ref[...]
    )


def kernel(x, adj, W1, b1, W2, b2):
    n, nfeat = x.shape
    nhid = W1.shape[1]
    nout = W2.shape[1]
    b1r = b1.reshape(1, nhid)
    b2r = b2.reshape(1, nout)

    s1 = pl.pallas_call(
        _s1_body,
        out_shape=jax.ShapeDtypeStruct((n, nhid), jnp.bfloat16),
    )(x, W1)

    grid = (pl.cdiv(n, _BR),)
    params = pltpu.CompilerParams(dimension_semantics=("parallel",))

    g = pl.pallas_call(
        _layer1_body,
        grid=grid,
        in_specs=[
            pl.BlockSpec((_BR, n), lambda r: (r, 0)),
            pl.BlockSpec((n, nhid), lambda r: (0, 0)),
            pl.BlockSpec((1, nhid), lambda r: (0, 0)),
            pl.BlockSpec((nhid, nout), lambda r: (0, 0)),
        ],
        out_specs=pl.BlockSpec((_BR, nout), lambda r: (r, 0)),
        out_shape=jax.ShapeDtypeStruct((n, nout), jnp.bfloat16),
        compiler_params=params,
    )(adj, s1, b1r, W2.astype(jnp.bfloat16))

    out = pl.pallas_call(
        _layer2_body,
        grid=grid,
        in_specs=[
            pl.BlockSpec((_BR, n), lambda r: (r, 0)),
            pl.BlockSpec((n, nout), lambda r: (0, 0)),
            pl.BlockSpec((1, nout), lambda r: (0, 0)),
        ],
        out_specs=pl.BlockSpec((_BR, nout), lambda r: (r, 0)),
        out_shape=jax.ShapeDtypeStruct((n, nout), jnp.float32),
        compiler_params=params,
    )(adj, g, b2r)

    return out
